# baseline (speedup 1.0000x reference)
import functools

import jax
import jax.numpy as jnp
from jax import lax
from jax.experimental import pallas as pl
from jax.experimental.pallas import tpu as pltpu
from jax.experimental.pallas import tpu_sc as plsc

B = 128
H = 256
V = 100000
K = 10

BV = 2048
NB = 49
VP = NB * BV
CH = 128
CPB = BV // CH
NCH = NB * CPB
NCHP = NB * 128

_SC_CORES = 2
_SC_SUBCORES = 16
_NW = _SC_CORES * _SC_SUBCORES
_ROWS_PER_W = (B * K) // _NW


def _stage_a_body(state_ref, w_ref, b_ref, logits_ref, m_ref, s_ref,
                  gidx_ref, glist_ref, cmax_ref):
    i = pl.program_id(0)

    @pl.when(i == 0)
    def _init():
        m_ref[...] = jnp.full((B, 1), -jnp.inf, jnp.float32)
        s_ref[...] = jnp.zeros((B, 1), jnp.float32)

    x = lax.dot_general(state_ref[...], w_ref[...],
                        (((1,), (1,)), ((), ())),
                        preferred_element_type=jnp.float32)
    x = x + b_ref[...][None, :]
    col = i * BV + lax.broadcasted_iota(jnp.int32, (B, BV), 1)
    x = jnp.where(col < V, x, -jnp.inf)
    logits_ref[...] = x.reshape(B // 8, 8, CPB, CH).transpose(0, 2, 1, 3)

    cm = jnp.max(x.reshape(B, CPB, CH), axis=2)
    pad = jnp.full((B, 128 - CPB), -jnp.inf, jnp.float32)
    cmax_ref[:, pl.ds(i * 128, 128)] = jnp.concatenate([cm, pad], axis=1)

    bm = jnp.max(x, axis=1, keepdims=True)
    m_old = m_ref[...]
    m_new = jnp.maximum(m_old, bm)
    s_ref[...] = (s_ref[...] * jnp.exp(m_old - m_new)
                  + jnp.sum(jnp.exp(x - m_new), axis=1, keepdims=True))
    m_ref[...] = m_new

    @pl.when(i == NB - 1)
    def _select_chunks():
        cm_all = cmax_ref[...].reshape(B, NB, 128)[:, :, :CPB].reshape(
            B, NCH)
        lane = lax.broadcasted_iota(jnp.int32, (B, NCH), 1)
        row = lax.broadcasted_iota(jnp.int32, (B, 1), 0)
        for k in range(K):
            mx = jnp.max(cm_all, axis=1, keepdims=True)
            sel = jnp.min(jnp.where(cm_all == mx, lane, NCH), axis=1,
                          keepdims=True)
            gr = (row >> 3) * (NCH * 8) + (sel << 3) + (row & 7)
            gidx_ref[:, k:k + 1] = gr
            glist_ref[pl.ds(k * B, B)] = gr.reshape(B)
            cm_all = jnp.where(lane == sel, -jnp.inf, cm_all)


_stage_a = pl.pallas_call(
    _stage_a_body,
    grid=(NB,),
    in_specs=[
        pl.BlockSpec((B, H), lambda i: (0, 0)),
        pl.BlockSpec((BV, H), lambda i: (i, 0)),
        pl.BlockSpec((BV,), lambda i: (i,)),
    ],
    out_specs=[
        pl.BlockSpec((B // 8, CPB, 8, CH), lambda i: (0, i, 0, 0)),
        pl.BlockSpec((B, 1), lambda i: (0, 0)),
        pl.BlockSpec((B, 1), lambda i: (0, 0)),
        pl.BlockSpec((B, K), lambda i: (0, 0)),
        pl.BlockSpec((B * K,), lambda i: (0,)),
    ],
    out_shape=[
        jax.ShapeDtypeStruct((B // 8, NCH, 8, CH), jnp.float32),
        jax.ShapeDtypeStruct((B, 1), jnp.float32),
        jax.ShapeDtypeStruct((B, 1), jnp.float32),
        jax.ShapeDtypeStruct((B, K), jnp.int32),
        jax.ShapeDtypeStruct((B * K,), jnp.int32),
    ],
    scratch_shapes=[pltpu.VMEM((B, NCHP), jnp.float32)],
)


def _sc_gather_body(table_hbm, idx_hbm, out_hbm, idx_v, rows_v, sem):
    wid = lax.axis_index("s") * _SC_CORES + lax.axis_index("c")
    base = wid * _ROWS_PER_W
    pltpu.sync_copy(idx_hbm.at[pl.ds(base, _ROWS_PER_W)], idx_v)
    pltpu.async_copy(table_hbm.at[idx_v], rows_v, sem).wait()
    pltpu.sync_copy(rows_v, out_hbm.at[pl.ds(base, _ROWS_PER_W)])


@functools.cache
def _sc_gather():
    return pl.kernel(
        _sc_gather_body,
        out_type=jax.ShapeDtypeStruct((B * K, CH), jnp.float32),
        mesh=plsc.VectorSubcoreMesh(
            core_axis_name="c", subcore_axis_name="s",
            num_cores=_SC_CORES, num_subcores=_SC_SUBCORES),
        scratch_types=[
            pltpu.VMEM((_ROWS_PER_W,), jnp.int32),
            pltpu.VMEM((_ROWS_PER_W, CH), jnp.float32),
            pltpu.SemaphoreType.DMA,
        ],
    )


def _stage_c_body(cand_ref, gidx_ref, m_ref, s_ref, rec_ref):
    row = lax.broadcasted_iota(jnp.int32, (B, K), 0)
    chunk = (gidx_ref[...] - (row >> 3) * (NCH * 8) - (row & 7)) >> 3
    base4 = (chunk * CH).T.reshape(K, B // 8, 8, 1)
    offs = lax.broadcasted_iota(jnp.int32, (K, B // 8, 8, CH), 3)
    gcol = base4 + offs
    m4 = m_ref[...].reshape(1, B // 8, 8, 1)
    s4 = s_ref[...].reshape(1, B // 8, 8, 1)
    p = jnp.exp(cand_ref[...] - m4) / s4
    for k in range(K):
        mx = jnp.max(p, axis=(0, 3), keepdims=True)
        sel = jnp.min(jnp.where(p == mx, gcol, V), axis=(0, 3),
                      keepdims=True)
        rec_ref[k:k + 1, :] = sel.reshape(1, B)
        p = jnp.where(gcol == sel, jnp.float32(-1.0), p)


_stage_c = pl.pallas_call(
    _stage_c_body,
    out_shape=jax.ShapeDtypeStruct((K, B), jnp.int32),
)


def kernel(state, W, b):
    logits4, m, s, gidx, glist = _stage_a(state, W.T, b)
    gathered = _sc_gather()(logits4.reshape(B * NCH, CH), glist)
    return _stage_c(gathered.reshape(K, B // 8, 8, CH), gidx, m, s).T


# speedup vs baseline: 1.3053x; 1.3053x over previous
import functools

import jax
import jax.numpy as jnp
from jax import lax
from jax.experimental import pallas as pl
from jax.experimental.pallas import tpu as pltpu
from jax.experimental.pallas import tpu_sc as plsc

B = 128
H = 256
V = 100000
K = 10

BV = 8192
NB = 13
VP = NB * BV
CH = 128
CPB = BV // CH
NCH = NB * CPB
NCHP = NB * 128

_SC_CORES = 2
_SC_SUBCORES = 16
_NW = _SC_CORES * _SC_SUBCORES
_ROWS_PER_W = (B * K) // _NW


def _stage_a_body(state_ref, w_ref, b_ref, logits_ref, m_ref, s_ref,
                  gidx_ref, glist_ref, cmax_ref):
    i = pl.program_id(0)

    @pl.when(i == 0)
    def _init():
        m_ref[...] = jnp.full((B, 1), -jnp.inf, jnp.float32)
        s_ref[...] = jnp.zeros((B, 1), jnp.float32)

    x = lax.dot_general(state_ref[...], w_ref[...],
                        (((1,), (1,)), ((), ())),
                        preferred_element_type=jnp.float32)
    x = x + b_ref[...][None, :]
    col = i * BV + lax.broadcasted_iota(jnp.int32, (B, BV), 1)
    x = jnp.where(col < V, x, -jnp.inf)
    logits_ref[...] = x.reshape(B // 8, 8, CPB, CH).transpose(0, 2, 1, 3)

    cm = jnp.max(x.reshape(B, CPB, CH), axis=2)
    pad = jnp.full((B, 128 - CPB), -jnp.inf, jnp.float32)
    cmax_ref[:, pl.ds(i * 128, 128)] = jnp.concatenate([cm, pad], axis=1)

    bm = jnp.max(x, axis=1, keepdims=True)
    m_old = m_ref[...]
    m_new = jnp.maximum(m_old, bm)
    s_ref[...] = (s_ref[...] * jnp.exp(m_old - m_new)
                  + jnp.sum(jnp.exp(x - m_new), axis=1, keepdims=True))
    m_ref[...] = m_new

    @pl.when(i == NB - 1)
    def _select_chunks():
        cm_all = cmax_ref[...].reshape(B, NB, 128)[:, :, :CPB].reshape(
            B, NCH)
        lane = lax.broadcasted_iota(jnp.int32, (B, NCH), 1)
        row = lax.broadcasted_iota(jnp.int32, (B, 1), 0)
        for k in range(K):
            mx = jnp.max(cm_all, axis=1, keepdims=True)
            sel = jnp.min(jnp.where(cm_all == mx, lane, NCH), axis=1,
                          keepdims=True)
            gr = (row >> 3) * (NCH * 8) + (sel << 3) + (row & 7)
            gidx_ref[:, k:k + 1] = gr
            glist_ref[pl.ds(k * B, B)] = gr.reshape(B)
            cm_all = jnp.where(lane == sel, -jnp.inf, cm_all)


_stage_a = pl.pallas_call(
    _stage_a_body,
    grid=(NB,),
    in_specs=[
        pl.BlockSpec((B, H), lambda i: (0, 0)),
        pl.BlockSpec((BV, H), lambda i: (i, 0)),
        pl.BlockSpec((BV,), lambda i: (i,)),
    ],
    out_specs=[
        pl.BlockSpec((B // 8, CPB, 8, CH), lambda i: (0, i, 0, 0)),
        pl.BlockSpec((B, 1), lambda i: (0, 0)),
        pl.BlockSpec((B, 1), lambda i: (0, 0)),
        pl.BlockSpec((B, K), lambda i: (0, 0)),
        pl.BlockSpec((B * K,), lambda i: (0,)),
    ],
    out_shape=[
        jax.ShapeDtypeStruct((B // 8, NCH, 8, CH), jnp.float32),
        jax.ShapeDtypeStruct((B, 1), jnp.float32),
        jax.ShapeDtypeStruct((B, 1), jnp.float32),
        jax.ShapeDtypeStruct((B, K), jnp.int32),
        jax.ShapeDtypeStruct((B * K,), jnp.int32),
    ],
    scratch_shapes=[pltpu.VMEM((B, NCHP), jnp.float32)],
)


def _sc_gather_body(table_hbm, idx_hbm, out_hbm, idx_v, rows_v, sem):
    wid = lax.axis_index("s") * _SC_CORES + lax.axis_index("c")
    base = wid * _ROWS_PER_W
    pltpu.sync_copy(idx_hbm.at[pl.ds(base, _ROWS_PER_W)], idx_v)
    pltpu.async_copy(table_hbm.at[idx_v], rows_v, sem).wait()
    pltpu.sync_copy(rows_v, out_hbm.at[pl.ds(base, _ROWS_PER_W)])


@functools.cache
def _sc_gather():
    return pl.kernel(
        _sc_gather_body,
        out_type=jax.ShapeDtypeStruct((B * K, CH), jnp.float32),
        mesh=plsc.VectorSubcoreMesh(
            core_axis_name="c", subcore_axis_name="s",
            num_cores=_SC_CORES, num_subcores=_SC_SUBCORES),
        scratch_types=[
            pltpu.VMEM((_ROWS_PER_W,), jnp.int32),
            pltpu.VMEM((_ROWS_PER_W, CH), jnp.float32),
            pltpu.SemaphoreType.DMA,
        ],
    )


def _stage_c_body(cand_ref, gidx_ref, m_ref, s_ref, rec_ref):
    row = lax.broadcasted_iota(jnp.int32, (B, K), 0)
    chunk = (gidx_ref[...] - (row >> 3) * (NCH * 8) - (row & 7)) >> 3
    base4 = (chunk * CH).T.reshape(K, B // 8, 8, 1)
    offs = lax.broadcasted_iota(jnp.int32, (K, B // 8, 8, CH), 3)
    gcol = base4 + offs
    m4 = m_ref[...].reshape(1, B // 8, 8, 1)
    s4 = s_ref[...].reshape(1, B // 8, 8, 1)
    p = jnp.exp(cand_ref[...] - m4) / s4
    for k in range(K):
        mx = jnp.max(p, axis=(0, 3), keepdims=True)
        sel = jnp.min(jnp.where(p == mx, gcol, V), axis=(0, 3),
                      keepdims=True)
        rec_ref[k:k + 1, :] = sel.reshape(1, B)
        p = jnp.where(gcol == sel, jnp.float32(-1.0), p)


_stage_c = pl.pallas_call(
    _stage_c_body,
    out_shape=jax.ShapeDtypeStruct((K, B), jnp.int32),
)


def kernel(state, W, b):
    logits4, m, s, gidx, glist = _stage_a(state, W.T, b)
    gathered = _sc_gather()(logits4.reshape(B * NCH, CH), glist)
    return _stage_c(gathered.reshape(K, B // 8, 8, CH), gidx, m, s).T


# speedup vs baseline: 1.3496x; 1.0339x over previous
import functools

import jax
import jax.numpy as jnp
from jax import lax
from jax.experimental import pallas as pl
from jax.experimental.pallas import tpu as pltpu
from jax.experimental.pallas import tpu_sc as plsc

B = 128
H = 256
V = 100000
K = 10

BV = 12800
NB = 8
VP = NB * BV
CH = 128
CPB = BV // CH
NCH = NB * CPB
NCHP = NB * 128

_SC_CORES = 2
_SC_SUBCORES = 16
_NW = _SC_CORES * _SC_SUBCORES
_ROWS_PER_W = (B * K) // _NW


def _stage_a_body(state_ref, w_ref, b_ref, logits_ref, m_ref, s_ref,
                  gidx_ref, glist_ref, cmax_ref):
    i = pl.program_id(0)

    @pl.when(i == 0)
    def _init():
        m_ref[...] = jnp.full((B, 1), -jnp.inf, jnp.float32)
        s_ref[...] = jnp.zeros((B, 1), jnp.float32)

    x = lax.dot_general(state_ref[...], w_ref[...],
                        (((1,), (1,)), ((), ())),
                        preferred_element_type=jnp.float32)
    x = x + b_ref[pl.ds(i * BV, BV)][None, :]
    col = i * BV + lax.broadcasted_iota(jnp.int32, (B, BV), 1)
    x = jnp.where(col < V, x, -jnp.inf)
    logits_ref[...] = x.reshape(B // 8, 8, CPB, CH).transpose(0, 2, 1, 3)

    cm = jnp.max(x.reshape(B, CPB, CH), axis=2)
    pad = jnp.full((B, 128 - CPB), -jnp.inf, jnp.float32)
    cmax_ref[:, pl.ds(i * 128, 128)] = jnp.concatenate([cm, pad], axis=1)

    bm = jnp.max(x, axis=1, keepdims=True)
    m_old = m_ref[...]
    m_new = jnp.maximum(m_old, bm)
    s_ref[...] = (s_ref[...] * jnp.exp(m_old - m_new)
                  + jnp.sum(jnp.exp(x - m_new), axis=1, keepdims=True))
    m_ref[...] = m_new

    @pl.when(i == NB - 1)
    def _select_chunks():
        cm_all = cmax_ref[...].reshape(B, NB, 128)[:, :, :CPB].reshape(
            B, NCH)
        lane = lax.broadcasted_iota(jnp.int32, (B, NCH), 1)
        row = lax.broadcasted_iota(jnp.int32, (B, 1), 0)
        for k in range(K):
            mx = jnp.max(cm_all, axis=1, keepdims=True)
            sel = jnp.min(jnp.where(cm_all == mx, lane, NCH), axis=1,
                          keepdims=True)
            gr = (row >> 3) * (NCH * 8) + (sel << 3) + (row & 7)
            gidx_ref[:, k:k + 1] = gr
            glist_ref[pl.ds(k * B, B)] = gr.reshape(B)
            cm_all = jnp.where(lane == sel, -jnp.inf, cm_all)


_stage_a = pl.pallas_call(
    _stage_a_body,
    grid=(NB,),
    in_specs=[
        pl.BlockSpec((B, H), lambda i: (0, 0)),
        pl.BlockSpec((BV, H), lambda i: (i, 0)),
        pl.BlockSpec((V,), lambda i: (0,)),
    ],
    out_specs=[
        pl.BlockSpec((B // 8, CPB, 8, CH), lambda i: (0, i, 0, 0)),
        pl.BlockSpec((B, 1), lambda i: (0, 0)),
        pl.BlockSpec((B, 1), lambda i: (0, 0)),
        pl.BlockSpec((B, K), lambda i: (0, 0)),
        pl.BlockSpec((B * K,), lambda i: (0,)),
    ],
    out_shape=[
        jax.ShapeDtypeStruct((B // 8, NCH, 8, CH), jnp.float32),
        jax.ShapeDtypeStruct((B, 1), jnp.float32),
        jax.ShapeDtypeStruct((B, 1), jnp.float32),
        jax.ShapeDtypeStruct((B, K), jnp.int32),
        jax.ShapeDtypeStruct((B * K,), jnp.int32),
    ],
    scratch_shapes=[pltpu.VMEM((B, NCHP), jnp.float32)],
)


def _sc_gather_body(table_hbm, idx_hbm, out_hbm, idx_v, rows_v, sem):
    wid = lax.axis_index("s") * _SC_CORES + lax.axis_index("c")
    base = wid * _ROWS_PER_W
    pltpu.sync_copy(idx_hbm.at[pl.ds(base, _ROWS_PER_W)], idx_v)
    pltpu.async_copy(table_hbm.at[idx_v], rows_v, sem).wait()
    pltpu.sync_copy(rows_v, out_hbm.at[pl.ds(base, _ROWS_PER_W)])


@functools.cache
def _sc_gather():
    return pl.kernel(
        _sc_gather_body,
        out_type=jax.ShapeDtypeStruct((B * K, CH), jnp.float32),
        mesh=plsc.VectorSubcoreMesh(
            core_axis_name="c", subcore_axis_name="s",
            num_cores=_SC_CORES, num_subcores=_SC_SUBCORES),
        scratch_types=[
            pltpu.VMEM((_ROWS_PER_W,), jnp.int32),
            pltpu.VMEM((_ROWS_PER_W, CH), jnp.float32),
            pltpu.SemaphoreType.DMA,
        ],
    )


def _stage_c_body(cand_ref, gidx_ref, m_ref, s_ref, rec_ref):
    row = lax.broadcasted_iota(jnp.int32, (B, K), 0)
    chunk = (gidx_ref[...] - (row >> 3) * (NCH * 8) - (row & 7)) >> 3
    base4 = (chunk * CH).T.reshape(K, B // 8, 8, 1)
    offs = lax.broadcasted_iota(jnp.int32, (K, B // 8, 8, CH), 3)
    gcol = base4 + offs
    m4 = m_ref[...].reshape(1, B // 8, 8, 1)
    s4 = s_ref[...].reshape(1, B // 8, 8, 1)
    p = jnp.exp(cand_ref[...] - m4) / s4
    for k in range(K):
        mx = jnp.max(p, axis=(0, 3), keepdims=True)
        sel = jnp.min(jnp.where(p == mx, gcol, V), axis=(0, 3),
                      keepdims=True)
        rec_ref[k:k + 1, :] = sel.reshape(1, B)
        p = jnp.where(gcol == sel, jnp.float32(-1.0), p)


_stage_c = pl.pallas_call(
    _stage_c_body,
    out_shape=jax.ShapeDtypeStruct((K, B), jnp.int32),
)


def kernel(state, W, b):
    logits4, m, s, gidx, glist = _stage_a(state, W.T, b)
    gathered = _sc_gather()(logits4.reshape(B * NCH, CH), glist)
    return _stage_c(gathered.reshape(K, B // 8, 8, CH), gidx, m, s).T
